# baseline (speedup 1.0000x reference)
import functools

import jax
import jax.numpy as jnp
from jax import lax
from jax.experimental import pallas as pl
from jax.experimental.pallas import tpu as pltpu
from jax.experimental.pallas import tpu_sc as plsc

VOCAB = 1000000
BATCH = 4096
SEQ = 200
D = 64
TOT = BATCH * SEQ
NW = 32
CHUNK = 128
NCH = TOT // CHUNK
CPW = NCH // NW
BT = BATCH // 128

_mesh = plsc.VectorSubcoreMesh(core_axis_name="c", subcore_axis_name="s")

PAN = 400
NPAN = VOCAB // PAN


@functools.partial(
    pl.kernel,
    mesh=_mesh,
    out_type=jax.ShapeDtypeStruct((2 * VOCAB, 32), jnp.float32),
    scratch_types=[
        pltpu.VMEM((D, PAN + 1), jnp.float32),
        pltpu.VMEM((D, PAN + 1), jnp.float32),
        pltpu.VMEM((2 * PAN, 32), jnp.float32),
        pltpu.VMEM((2 * PAN, 32), jnp.float32),
        pltpu.SemaphoreType.DMA,
        pltpu.SemaphoreType.DMA,
    ],
    compiler_params=pltpu.CompilerParams(
        use_tc_tiling_on_sc=False, needs_layout_passes=False
    ),
)
def _sc_tab_t(tcm_hbm, tt_hbm, pan0, pan1, ob0, ob1, is0, is1):
    wid = lax.axis_index("s") * 2 + lax.axis_index("c")
    npan = 78 + jnp.where(wid < NPAN - 78 * NW, 1, 0)
    pans = (pan0, pan1)
    obs = (ob0, ob1)
    isem = (is0, is1)

    iota = lax.iota(jnp.int32, 16)
    rvec = [iota + (32 * hh + 16 * cb) for hh in range(2) for cb in range(2)]

    def pid(j):
        return wid + NW * j

    def fire_in(j, p):
        pltpu.async_copy(
            tcm_hbm.at[:, pl.ds(PAN * pid(j), PAN)],
            pans[p].at[:, pl.ds(0, PAN)],
            isem[p],
        )

    def wait_in(p):
        pltpu.make_async_copy(
            tcm_hbm.at[:, pl.ds(0, PAN)],
            pans[p].at[:, pl.ds(0, PAN)],
            isem[p],
        ).wait()

    def trans_panel(p):
        pan, ob = pans[p], obs[p]

        def body(v, carry):
            cv = jnp.full((16,), v, dtype=jnp.int32)
            vals = [plsc.load_gather(pan, [rvec[i], cv]) for i in range(4)]
            for hh in range(2):
                for cb in range(2):
                    ob[2 * v + hh, pl.ds(16 * cb, 16)] = vals[hh * 2 + cb]
            return carry

        lax.fori_loop(0, PAN, body, 0)

    def step(j, p, first):
        if first:
            wait_in(p)
            fire_in(j + 1, 1 - p)
            trans_panel(p)
            pltpu.sync_copy(
                obs[p], tt_hbm.at[pl.ds(2 * PAN * pid(j), 2 * PAN)]
            )
            return

        @pl.when(j < npan)
        def _():
            wait_in(p)

            @pl.when(j + 1 < npan)
            def _():
                fire_in(j + 1, 1 - p)

            trans_panel(p)
            pltpu.sync_copy(
                obs[p], tt_hbm.at[pl.ds(2 * PAN * pid(j), 2 * PAN)]
            )

    fire_in(0, 0)
    step(0, 0, True)

    def loop_body(i, carry):
        step(2 * i + 1, 1, False)
        step(2 * i + 2, 0, False)
        return carry

    lax.fori_loop(0, 40, loop_body, 0)


@functools.partial(
    pl.kernel,
    mesh=_mesh,
    out_type=jax.ShapeDtypeStruct((SEQ, D // 8, BT, 8, 128), jnp.float32),
    scratch_types=[
        pltpu.VMEM((CPW, 2, 128), jnp.int32),
        pltpu.VMEM((2 * CHUNK, 32), jnp.float32),
        pltpu.VMEM((2 * CHUNK, 32), jnp.float32),
        pltpu.VMEM((D, 128), jnp.float32),
        pltpu.VMEM((D, 128), jnp.float32),
        pltpu.SemaphoreType.DMA,
        pltpu.SemaphoreType.DMA,
        pltpu.SemaphoreType.DMA,
        pltpu.SemaphoreType.DMA,
    ],
    compiler_params=pltpu.CompilerParams(
        use_tc_tiling_on_sc=False, needs_layout_passes=False
    ),
)
def _sc_embed(tab_hbm, idx_hbm, out_hbm, idx_v, g0, g1, t0, t1,
              gs0, gs1, ws0, ws1):
    wid = lax.axis_index("s") * 2 + lax.axis_index("c")
    base = wid * CPW
    gbuf = (g0, g1)
    tbuf = (t0, t1)
    gsem = (gs0, gs1)
    wsem = (ws0, ws1)

    pltpu.sync_copy(idx_hbm.at[wid], idx_v)

    iota = lax.iota(jnp.int32, 16)
    rbase = [2 * iota + 32 * k for k in range(8)]

    def fire_gather(j, p):
        for h in range(2):
            pltpu.async_copy(
                tab_hbm.at[idx_v.at[j, h]],
                gbuf[p].at[pl.ds(128 * h, 128)],
                gsem[p],
            )

    def wait_gather(p):
        for h in range(2):
            pltpu.make_async_copy(
                tab_hbm.at[idx_v.at[0, 0]],
                gbuf[p].at[pl.ds(128 * h, 128)],
                gsem[p],
            ).wait()

    def transpose_chunk(p):
        g, t = gbuf[p], tbuf[p]
        for half in range(2):
            rows = [rbase[k] + half for k in range(8)]

            def body(i, carry, rows=rows, half=half):
                dl = i * 2
                c0 = jnp.full((16,), dl, dtype=jnp.int32)
                c1 = c0 + 1
                vs0 = [plsc.load_gather(g, [rows[k], c0]) for k in range(8)]
                vs1 = [plsc.load_gather(g, [rows[k], c1]) for k in range(8)]
                d = 32 * half + dl
                for k in range(8):
                    t[d, pl.ds(16 * k, 16)] = vs0[k]
                for k in range(8):
                    t[d + 1, pl.ds(16 * k, 16)] = vs1[k]
                return carry

            lax.fori_loop(0, 16, body, 0)

    def fire_write(j, p):
        c = base + j
        s = c // BT
        bt = c % BT
        for dt in range(8):
            pltpu.async_copy(
                tbuf[p].at[pl.ds(8 * dt, 8)],
                out_hbm.at[s, dt, bt],
                wsem[p],
            )

    def wait_write(p):
        for dt in range(8):
            pltpu.make_async_copy(
                tbuf[p].at[pl.ds(8 * dt, 8)],
                out_hbm.at[0, dt, 0],
                wsem[p],
            ).wait()

    def step(j, p, first):
        wait_gather(p)

        if first:
            fire_gather(j + 1, 1 - p)
        else:
            @pl.when(j + 1 < CPW)
            def _():
                fire_gather(j + 1, 1 - p)
            wait_write(p)
        transpose_chunk(p)
        fire_write(j, p)

    fire_gather(0, 0)
    step(0, 0, True)
    step(1, 1, True)

    def loop_body(i, carry):
        step(2 * i, 0, False)
        step(2 * i + 1, 1, False)
        return carry

    lax.fori_loop(1, CPW // 2, loop_body, 0)
    wait_write(0)
    wait_write(1)


def kernel(inp, table):
    idx_t = jnp.transpose(inp).reshape(NCH, CHUNK)
    idx2 = (2 * idx_t)[:, :, None] + jnp.arange(2, dtype=inp.dtype)
    idx4 = idx2.reshape(NW, CPW, 2, 128)
    out5 = _sc_embed(_sc_tab_t(jnp.transpose(table)), idx4)
    return out5.transpose(0, 2, 4, 1, 3).reshape(SEQ, BATCH, D)


# speedup vs baseline: 6.9591x; 6.9591x over previous
import functools

import jax
import jax.numpy as jnp
from jax import lax
from jax.experimental import pallas as pl
from jax.experimental.pallas import tpu as pltpu
from jax.experimental.pallas import tpu_sc as plsc

VOCAB = 1000000
BATCH = 4096
SEQ = 200
D = 64
TOT = BATCH * SEQ
NW = 32
CHUNK = 128
NCH = TOT // CHUNK
CPW = NCH // NW
BT = BATCH // 128

_mesh = plsc.VectorSubcoreMesh(core_axis_name="c", subcore_axis_name="s")

@functools.partial(
    pl.kernel,
    mesh=_mesh,
    out_type=jax.ShapeDtypeStruct((SEQ, D // 8, BT, 8, 128), jnp.float32),
    scratch_types=[
        pltpu.VMEM((CPW, 2, 128), jnp.int32),
        pltpu.VMEM((2 * CHUNK, 32), jnp.float32),
        pltpu.VMEM((2 * CHUNK, 32), jnp.float32),
        pltpu.VMEM((D, 129), jnp.float32),
        pltpu.VMEM((D, 129), jnp.float32),
        pltpu.SemaphoreType.DMA,
        pltpu.SemaphoreType.DMA,
        pltpu.SemaphoreType.DMA,
        pltpu.SemaphoreType.DMA,
    ],
    compiler_params=pltpu.CompilerParams(
        use_tc_tiling_on_sc=False, needs_layout_passes=False
    ),
)
def _sc_embed(tab_hbm, idx_hbm, out_hbm, idx_v, g0, g1, t0, t1,
              gs0, gs1, ws0, ws1):
    wid = lax.axis_index("s") * 2 + lax.axis_index("c")
    base = wid * CPW
    gbuf = (g0, g1)
    tbuf = (t0, t1)
    gsem = (gs0, gs1)
    wsem = (ws0, ws1)

    pltpu.sync_copy(idx_hbm.at[wid], idx_v)

    iota = lax.iota(jnp.int32, 16)
    rscat = [iota + 32 * hh + 16 * q for hh in range(2) for q in range(2)]

    def fire_gather(j, p):
        for h in range(2):
            pltpu.async_copy(
                tab_hbm.at[idx_v.at[j, h]],
                gbuf[p].at[pl.ds(128 * h, 128)],
                gsem[p],
            )

    def wait_gather(p):
        for h in range(2):
            pltpu.make_async_copy(
                tab_hbm.at[idx_v.at[0, 0]],
                gbuf[p].at[pl.ds(128 * h, 128)],
                gsem[p],
            ).wait()

    def transpose_chunk(p):
        g, t = gbuf[p], tbuf[p]

        def body(b, carry):
            cb = jnp.full((16,), b, dtype=jnp.int32)
            for hh in range(2):
                r = 2 * b + hh
                v0 = g[r, pl.ds(0, 16)]
                v1 = g[r, pl.ds(16, 16)]
                plsc.store_scatter(t, [rscat[hh * 2], cb], v0)
                plsc.store_scatter(t, [rscat[hh * 2 + 1], cb], v1)
            return carry

        lax.fori_loop(0, CHUNK, body, 0)

    def fire_write(j, p):
        c = base + j
        s = c // BT
        bt = c % BT
        for dt in range(8):
            pltpu.async_copy(
                tbuf[p].at[pl.ds(8 * dt, 8), pl.ds(0, 128)],
                out_hbm.at[s, dt, bt],
                wsem[p],
            )

    def wait_write(p):
        for dt in range(8):
            pltpu.make_async_copy(
                tbuf[p].at[pl.ds(8 * dt, 8), pl.ds(0, 128)],
                out_hbm.at[0, dt, 0],
                wsem[p],
            ).wait()

    def step(j, p, first):
        wait_gather(p)

        if first:
            fire_gather(j + 1, 1 - p)
        else:
            @pl.when(j + 1 < CPW)
            def _():
                fire_gather(j + 1, 1 - p)
            wait_write(p)
        transpose_chunk(p)
        fire_write(j, p)

    fire_gather(0, 0)
    step(0, 0, True)
    step(1, 1, True)

    def loop_body(i, carry):
        step(2 * i, 0, False)
        step(2 * i + 1, 1, False)
        return carry

    lax.fori_loop(1, CPW // 2, loop_body, 0)
    wait_write(0)
    wait_write(1)


def kernel(inp, table):
    idx_t = jnp.transpose(inp).reshape(NCH, CHUNK)
    idx2 = (2 * idx_t)[:, :, None] + jnp.arange(2, dtype=inp.dtype)
    idx4 = idx2.reshape(NW, CPW, 2, 128)
    out5 = _sc_embed(table.reshape(2 * VOCAB, 32), idx4)
    return out5.transpose(0, 2, 4, 1, 3).reshape(SEQ, BATCH, D)
